# Initial kernel scaffold; baseline (speedup 1.0000x reference)
import jax
import jax.numpy as jnp
from jax.experimental import pallas as pl


def kernel(x, W, b_enc, b_dec):
    raise NotImplementedError("write your pallas kernel here")


# speedup vs baseline: 8.4342x; 8.4342x over previous
import functools

import jax
import jax.numpy as jnp
from jax.experimental import pallas as pl
from jax.experimental.pallas import tpu as pltpu

K = 64


def _enc_body(x_ref, w_ref, be_ref, out_ref):
    acc = jax.lax.dot_general(
        x_ref[...], w_ref[...],
        dimension_numbers=(((1,), (1,)), ((), ())),
        preferred_element_type=jnp.float32,
    )
    out_ref[...] = acc + be_ref[...][None, :]


def _encode(x, W, b_enc, br, bl):
    n, d_in = x.shape
    d_lat = W.shape[0]
    grid = (d_lat // bl, n // br)
    return pl.pallas_call(
        _enc_body,
        grid=grid,
        in_specs=[
            pl.BlockSpec((br, d_in), lambda i, j: (j, 0)),
            pl.BlockSpec((bl, d_in), lambda i, j: (i, 0)),
            pl.BlockSpec((bl,), lambda i, j: (i,)),
        ],
        out_specs=pl.BlockSpec((br, bl), lambda i, j: (j, i)),
        out_shape=jax.ShapeDtypeStruct((n, d_lat), jnp.float32),
    )(x, W, b_enc)


def _select_body(pre_ref, z_ref):
    q = jnp.maximum(pre_ref[...], 0.0)
    s = jax.lax.bitcast_convert_type(q, jnp.int32)
    t = jnp.zeros((q.shape[0], 1), dtype=jnp.int32)
    for b in range(30, -1, -1):
        cand = t | (1 << b)
        cnt = jnp.sum((s >= cand).astype(jnp.int32), axis=1, keepdims=True)
        t = jnp.where(cnt >= K, cand, t)
    mask = s >= jnp.maximum(t, 1)
    z_ref[...] = jnp.where(mask, q, 0.0)


def _select(pre, br):
    n, d_lat = pre.shape
    return pl.pallas_call(
        _select_body,
        grid=(n // br,),
        in_specs=[pl.BlockSpec((br, d_lat), lambda i: (i, 0))],
        out_specs=pl.BlockSpec((br, d_lat), lambda i: (i, 0)),
        out_shape=jax.ShapeDtypeStruct((n, d_lat), jnp.float32),
    )(pre)


def _dec_body(z_ref, w_ref, bd_ref, out_ref):
    l = pl.program_id(1)
    acc = jax.lax.dot_general(
        z_ref[...].astype(jnp.bfloat16), w_ref[...],
        dimension_numbers=(((1,), (0,)), ((), ())),
        preferred_element_type=jnp.float32,
    )

    @pl.when(l == 0)
    def _():
        out_ref[...] = acc + bd_ref[...][None, :]

    @pl.when(l > 0)
    def _():
        out_ref[...] += acc


def _decode(z, W_bf16, b_dec, br, lt):
    n, d_lat = z.shape
    d_in = W_bf16.shape[1]
    grid = (n // br, d_lat // lt)
    return pl.pallas_call(
        _dec_body,
        grid=grid,
        in_specs=[
            pl.BlockSpec((br, lt), lambda i, l: (i, l)),
            pl.BlockSpec((lt, d_in), lambda i, l: (l, 0)),
            pl.BlockSpec((d_in,), lambda i, l: (0,)),
        ],
        out_specs=pl.BlockSpec((br, d_in), lambda i, l: (i, 0)),
        out_shape=jax.ShapeDtypeStruct((n, d_in), jnp.float32),
    )(z, W_bf16, b_dec)


@functools.partial(jax.jit, static_argnames=())
def kernel(x, W, b_enc, b_dec):
    pre = _encode(x, W, b_enc, br=256, bl=1024)
    z = _select(pre, br=128)
    recon = _decode(z, W.astype(jnp.bfloat16), b_dec, br=512, lt=1024)
    return (recon, z)


# speedup vs baseline: 8.4751x; 1.0049x over previous
import functools

import jax
import jax.numpy as jnp
from jax.experimental import pallas as pl
from jax.experimental.pallas import tpu as pltpu

K = 64


def _enc_body(x_ref, w_ref, be_ref, out_ref):
    acc = jax.lax.dot_general(
        x_ref[...], w_ref[...],
        dimension_numbers=(((1,), (1,)), ((), ())),
        preferred_element_type=jnp.float32,
    )
    out_ref[...] = acc + be_ref[...][None, :]


def _encode(x, W, b_enc, br, bl):
    n, d_in = x.shape
    d_lat = W.shape[0]
    grid = (d_lat // bl, n // br)
    return pl.pallas_call(
        _enc_body,
        grid=grid,
        in_specs=[
            pl.BlockSpec((br, d_in), lambda i, j: (j, 0)),
            pl.BlockSpec((bl, d_in), lambda i, j: (i, 0)),
            pl.BlockSpec((bl,), lambda i, j: (i,)),
        ],
        out_specs=pl.BlockSpec((br, bl), lambda i, j: (j, i)),
        out_shape=jax.ShapeDtypeStruct((n, d_lat), jnp.float32),
    )(x, W, b_enc)


def _select_body(pre_ref, z_ref):
    q = jnp.maximum(pre_ref[...], 0.0)
    s = jax.lax.bitcast_convert_type(q, jnp.int32)
    t = jnp.zeros((q.shape[0], 1), dtype=jnp.int32)
    for b in range(30, -1, -1):
        cand = t | (1 << b)
        cnt = jnp.sum((s >= cand).astype(jnp.int32), axis=1, keepdims=True)
        t = jnp.where(cnt >= K, cand, t)
    mask = s >= jnp.maximum(t, 1)
    z_ref[...] = jnp.where(mask, q, 0.0)


def _select(pre, br):
    n, d_lat = pre.shape
    return pl.pallas_call(
        _select_body,
        grid=(n // br,),
        in_specs=[pl.BlockSpec((br, d_lat), lambda i: (i, 0))],
        out_specs=pl.BlockSpec((br, d_lat), lambda i: (i, 0)),
        out_shape=jax.ShapeDtypeStruct((n, d_lat), jnp.float32),
    )(pre)


def _dec_body(z_ref, w_ref, bd_ref, out_ref):
    l = pl.program_id(1)
    acc = jax.lax.dot_general(
        z_ref[...].astype(jnp.bfloat16), w_ref[...],
        dimension_numbers=(((1,), (0,)), ((), ())),
        preferred_element_type=jnp.float32,
    )

    @pl.when(l == 0)
    def _():
        out_ref[...] = acc + bd_ref[...][None, :]

    @pl.when(l > 0)
    def _():
        out_ref[...] += acc


def _decode(z, W_bf16, b_dec, br, lt):
    n, d_lat = z.shape
    d_in = W_bf16.shape[1]
    grid = (n // br, d_lat // lt)
    return pl.pallas_call(
        _dec_body,
        grid=grid,
        in_specs=[
            pl.BlockSpec((br, lt), lambda i, l: (i, l)),
            pl.BlockSpec((lt, d_in), lambda i, l: (l, 0)),
            pl.BlockSpec((d_in,), lambda i, l: (0,)),
        ],
        out_specs=pl.BlockSpec((br, d_in), lambda i, l: (i, 0)),
        out_shape=jax.ShapeDtypeStruct((n, d_in), jnp.float32),
    )(z, W_bf16, b_dec)


@functools.partial(jax.jit, static_argnames=())
def kernel(x, W, b_enc, b_dec):
    W_bf16 = W.astype(jnp.bfloat16)
    pre = _encode(x.astype(jnp.bfloat16), W_bf16, b_enc, br=256, bl=1024)
    z = _select(pre, br=128)
    recon = _decode(z, W_bf16, b_dec, br=512, lt=1024)
    return (recon, z)


# speedup vs baseline: 8.4824x; 1.0009x over previous
import functools

import jax
import jax.numpy as jnp
from jax.experimental import pallas as pl
from jax.experimental.pallas import tpu as pltpu

K = 64


def _enc_body(x_ref, w_ref, be_ref, out_ref):
    acc = jax.lax.dot_general(
        x_ref[...], w_ref[...],
        dimension_numbers=(((1,), (1,)), ((), ())),
        preferred_element_type=jnp.float32,
    )
    out_ref[...] = acc + be_ref[...][None, :]


def _encode(x, W, b_enc, br, bl):
    n, d_in = x.shape
    d_lat = W.shape[0]
    grid = (d_lat // bl, n // br)
    return pl.pallas_call(
        _enc_body,
        grid=grid,
        in_specs=[
            pl.BlockSpec((br, d_in), lambda i, j: (j, 0)),
            pl.BlockSpec((bl, d_in), lambda i, j: (i, 0)),
            pl.BlockSpec((bl,), lambda i, j: (i,)),
        ],
        out_specs=pl.BlockSpec((br, bl), lambda i, j: (j, i)),
        out_shape=jax.ShapeDtypeStruct((n, d_lat), jnp.float32),
    )(x, W, b_enc)


def _thresh_body(pre_ref, t_ref):
    q = jnp.maximum(pre_ref[...], 0.0)
    s = jax.lax.bitcast_convert_type(q, jnp.int32)
    t = jnp.zeros((q.shape[0], 1), dtype=jnp.int32)
    for b in range(30, -1, -1):
        cand = t | (1 << b)
        cnt = jnp.sum(s >= cand, axis=1, keepdims=True, dtype=jnp.int32)
        t = jnp.where(cnt >= K, cand, t)
    t_ref[...] = jnp.maximum(t[:, 0], 1)


def _thresholds(pre, br):
    n, d_lat = pre.shape
    return pl.pallas_call(
        _thresh_body,
        grid=(n // br,),
        in_specs=[pl.BlockSpec((br, d_lat), lambda i: (i, 0))],
        out_specs=pl.BlockSpec((br,), lambda i: (i,)),
        out_shape=jax.ShapeDtypeStruct((n,), jnp.int32),
    )(pre)


def _dec_body(pre_ref, t_ref, w_ref, bd_ref, out_ref, z_ref):
    l = pl.program_id(1)
    q = jnp.maximum(pre_ref[...], 0.0)
    s = jax.lax.bitcast_convert_type(q, jnp.int32)
    zb = jnp.where(s >= t_ref[...][:, None], q, 0.0)
    z_ref[...] = zb
    acc = jax.lax.dot_general(
        zb.astype(jnp.bfloat16), w_ref[...],
        dimension_numbers=(((1,), (0,)), ((), ())),
        preferred_element_type=jnp.float32,
    )

    @pl.when(l == 0)
    def _():
        out_ref[...] = acc + bd_ref[...][None, :]

    @pl.when(l > 0)
    def _():
        out_ref[...] += acc


def _decode(pre, t, W_bf16, b_dec, br, lt):
    n, d_lat = pre.shape
    d_in = W_bf16.shape[1]
    grid = (n // br, d_lat // lt)
    return pl.pallas_call(
        _dec_body,
        grid=grid,
        in_specs=[
            pl.BlockSpec((br, lt), lambda i, l: (i, l)),
            pl.BlockSpec((br,), lambda i, l: (i,)),
            pl.BlockSpec((lt, d_in), lambda i, l: (l, 0)),
            pl.BlockSpec((d_in,), lambda i, l: (0,)),
        ],
        out_specs=[
            pl.BlockSpec((br, d_in), lambda i, l: (i, 0)),
            pl.BlockSpec((br, lt), lambda i, l: (i, l)),
        ],
        out_shape=[
            jax.ShapeDtypeStruct((n, d_in), jnp.float32),
            jax.ShapeDtypeStruct((n, d_lat), jnp.float32),
        ],
    )(pre, t, W_bf16, b_dec)


@functools.partial(jax.jit, static_argnames=())
def kernel(x, W, b_enc, b_dec):
    W_bf16 = W.astype(jnp.bfloat16)
    pre = _encode(x.astype(jnp.bfloat16), W_bf16, b_enc, br=256, bl=1024)
    t = _thresholds(pre, br=128)
    recon, z = _decode(pre, t, W_bf16, b_dec, br=512, lt=1024)
    return (recon, z)
